# baseline (speedup 1.0000x reference)
import functools

import jax
import jax.numpy as jnp
from jax import lax
from jax.experimental import pallas as pl
from jax.experimental.pallas import tpu as pltpu
from jax.experimental.pallas import tpu_sc as plsc

NPTS = 10000
DIM = 128
KNN = 16
NTILES = 32
QG = 4
GPT = 79
NVREG = NPTS // 16
CHUNK_V = 156
NCHUNK = 4
CBUF = CHUNK_V * 16


def _mm_body(x_ref, w1_ref, b1_ref, p_ref, q_ref):
    x = x_ref[...]
    w1a = w1_ref[:DIM, :]
    w1b = w1_ref[DIM:, :]
    q_ref[...] = jnp.dot(x, w1b, preferred_element_type=jnp.float32)
    p_ref[...] = (
        jnp.dot(x, w1a - w1b, preferred_element_type=jnp.float32) + b1_ref[...]
    )


def _phase_a(x, w1, b1r):
    return pl.pallas_call(
        _mm_body,
        out_shape=(
            jax.ShapeDtypeStruct((NPTS, DIM), jnp.float32),
            jax.ShapeDtypeStruct((NPTS, DIM), jnp.float32),
        ),
    )(x, w1, b1r)


def _bf16_rne(v):
    u = plsc.bitcast(v, jnp.uint32)
    r = (u + jnp.uint32(0x7FFF) + ((u >> jnp.uint32(16)) & jnp.uint32(1)))
    r = r & jnp.uint32(0xFFFF0000)
    return plsc.bitcast(r, jnp.float32)


def _knn_body(pos_hbm, q_hbm, g_hbm,
              bx, by, bz, sq, pflat,
              cd0, cj0, cd1, cj1, cd2, cj2, cd3, cj3,
              gbuf, gsem):
    cid = lax.axis_index("c")
    sid = lax.axis_index("s")
    wid = sid * 2 + cid
    cands = ((cd0, cj0), (cd1, cj1), (cd2, cj2), (cd3, cj3))

    pltpu.sync_copy(pos_hbm, pflat)
    iota3 = lax.iota(jnp.int32, 16) * 3

    def stage_body(i, _):
        base = i * 48
        gx = plsc.load_gather(pflat, [iota3 + base])
        gy = plsc.load_gather(pflat, [iota3 + (base + 1)])
        gz = plsc.load_gather(pflat, [iota3 + (base + 2)])
        sq[pl.ds(i * 16, 16)] = gx * gx + gy * gy + gz * gz
        bx[pl.ds(i * 16, 16)] = _bf16_rne(gx)
        by[pl.ds(i * 16, 16)] = _bf16_rne(gy)
        bz[pl.ds(i * 16, 16)] = _bf16_rne(gz)
        return 0

    lax.fori_loop(0, NVREG, stage_body, 0)

    iota = lax.iota(jnp.int32, 16)
    inf_v = jnp.full((16,), jnp.inf, jnp.float32)
    zero_i = jnp.zeros((16,), jnp.int32)
    negone_i = jnp.full((16,), -1, jnp.int32)

    def group_body(gi, _):
        nodes = [jnp.minimum(wid * (GPT * QG) + gi * QG + q, NPTS - 1)
                 for q in range(QG)]
        xis, yis, zis = [], [], []
        for q in range(QG):
            xis.append(bx[pl.ds(nodes[q], 16)][0] * -2.0)
            yis.append(by[pl.ds(nodes[q], 16)][0] * -2.0)
            zis.append(bz[pl.ds(nodes[q], 16)][0] * -2.0)

        def chunk_body(ch, carry):
            st = list(carry)
            base0 = (ch * CHUNK_V + 1) * 16
            for v in range(CHUNK_V):
                b = base0 + v * 16
                sqv = sq[pl.ds(b, 16)]
                bxv = bx[pl.ds(b, 16)]
                byv = by[pl.ds(b, 16)]
                bzv = bz[pl.ds(b, 16)]
                jv = iota + b
                for q in range(QG):
                    mvec = st[4 * q + 2]
                    curs = st[4 * q + 3]
                    d = (sqv + bxv * xis[q]) + (byv * yis[q] + bzv * zis[q])
                    hit = d < mvec
                    ps = plsc.cumsum(hit.astype(jnp.int32))
                    posn = curs + ps
                    plsc.store_scatter(cands[q][0], [posn], d, mask=hit)
                    plsc.store_scatter(cands[q][1], [posn], jv, mask=hit)
                    st[4 * q + 3] = curs + ps[15]

            for q in range(QG):
                topk = st[4 * q]
                topv = st[4 * q + 1]
                cnt_s = st[4 * q + 3][0] + 1
                node = nodes[q]
                cd, cj = cands[q]

                def do_merge(c2, cnt_s=cnt_s, node=node, cd=cd, cj=cj):
                    nv = (cnt_s + 15) >> 4

                    def mbody(it, c3):
                        tk, tv = c3
                        mb = it * 16
                        dc = cd[pl.ds(mb, 16)]
                        jc = cj[pl.ds(mb, 16)]
                        valid = iota < (cnt_s - mb)
                        dc = jnp.where(valid, dc, jnp.inf)
                        dc = jnp.where(jc == node, jnp.inf, dc)
                        sk, sv = plsc.sort_key_val(dc, jc)
                        rk = jnp.flip(sk)
                        rv = jnp.flip(sv)
                        take = rk < tk
                        nk = jnp.where(take, rk, tk)
                        nvv = jnp.where(take, rv, tv)
                        tk2, tv2 = plsc.sort_key_val(nk, nvv)
                        return (tk2, tv2)

                    return lax.fori_loop(0, nv, mbody, c2)

                topk, topv = lax.cond(cnt_s > 0, do_merge, lambda c2: c2,
                                      (topk, topv))
                st[4 * q] = topk
                st[4 * q + 1] = topv
                st[4 * q + 2] = jnp.broadcast_to(topk[15], (16,))
                st[4 * q + 3] = negone_i
            return tuple(st)

        sq0 = sq[pl.ds(0, 16)]
        bx0 = bx[pl.ds(0, 16)]
        by0 = by[pl.ds(0, 16)]
        bz0 = bz[pl.ds(0, 16)]
        seed = []
        for q in range(QG):
            d0 = (sq0 + bx0 * xis[q]) + (by0 * yis[q] + bz0 * zis[q])
            d0 = jnp.where(iota == nodes[q], jnp.inf, d0)
            sk, sv = plsc.sort_key_val(d0, iota)
            seed += [sk, sv, jnp.broadcast_to(sk[15], (16,)), negone_i]
        fin = lax.fori_loop(0, NCHUNK, chunk_body, tuple(seed))

        gd = [pltpu.async_copy(q_hbm.at[fin[4 * q + 1]], gbuf.at[q], gsem)
              for q in range(QG)]
        for c in gd:
            c.wait()
        sd = [pltpu.async_copy(gbuf.at[q], g_hbm.at[pl.ds(nodes[q] * KNN, KNN)],
                               gsem) for q in range(QG)]
        for c in sd:
            c.wait()
        return 0

    lax.fori_loop(0, GPT, group_body, 0)


def _sc_knn_gather(pos_flat, q):
    mesh = plsc.VectorSubcoreMesh(core_axis_name="c", subcore_axis_name="s")
    kern = pl.kernel(
        _knn_body,
        out_type=jax.ShapeDtypeStruct((NPTS * KNN, DIM), jnp.float32),
        mesh=mesh,
        scratch_types=[
            pltpu.VMEM((NPTS + 16,), jnp.float32),
            pltpu.VMEM((NPTS + 16,), jnp.float32),
            pltpu.VMEM((NPTS + 16,), jnp.float32),
            pltpu.VMEM((NPTS,), jnp.float32),
            pltpu.VMEM((3 * NPTS,), jnp.float32),
            pltpu.VMEM((CBUF,), jnp.float32),
            pltpu.VMEM((CBUF,), jnp.int32),
            pltpu.VMEM((CBUF,), jnp.float32),
            pltpu.VMEM((CBUF,), jnp.int32),
            pltpu.VMEM((CBUF,), jnp.float32),
            pltpu.VMEM((CBUF,), jnp.int32),
            pltpu.VMEM((CBUF,), jnp.float32),
            pltpu.VMEM((CBUF,), jnp.int32),
            pltpu.VMEM((QG, KNN, DIM), jnp.float32),
            pltpu.SemaphoreType.DMA,
        ],
        compiler_params=pltpu.CompilerParams(needs_layout_passes=False),
    )
    return kern(pos_flat, q)


NB = 256


def _mlp_body(g_ref, p_ref, w2_ref, b2_ref, o_ref):
    p = p_ref[...]
    w2 = w2_ref[...]
    acc = None
    for s in range(KNN):
        h1 = jnp.maximum(p + g_ref[:, s, :], 0.0)
        h2 = jnp.dot(h1, w2, preferred_element_type=jnp.float32)
        acc = h2 if acc is None else jnp.maximum(acc, h2)
    o_ref[...] = acc + b2_ref[...]


def _phase_c(ge, p, w2, b2r):
    grid = (pl.cdiv(NPTS, NB),)
    return pl.pallas_call(
        _mlp_body,
        grid=grid,
        in_specs=[
            pl.BlockSpec((NB, KNN, DIM), lambda i: (i, 0, 0)),
            pl.BlockSpec((NB, DIM), lambda i: (i, 0)),
            pl.BlockSpec((DIM, DIM), lambda i: (0, 0)),
            pl.BlockSpec((1, DIM), lambda i: (0, 0)),
        ],
        out_specs=pl.BlockSpec((NB, DIM), lambda i: (i, 0)),
        out_shape=jax.ShapeDtypeStruct((NPTS, DIM), jnp.float32),
    )(ge, p, w2, b2r)


def kernel(x, pos, W1, b1, W2, b2):
    p, q = _phase_a(x, W1, b1.reshape(1, DIM))
    g = _sc_knn_gather(pos.reshape(3 * NPTS), q)
    ge = g.reshape(NPTS, KNN, DIM)
    return _phase_c(ge, p, W2, b2.reshape(1, DIM))


# speedup vs baseline: 2.7639x; 2.7639x over previous
import functools

import jax
import jax.numpy as jnp
from jax import lax
from jax.experimental import pallas as pl
from jax.experimental.pallas import tpu as pltpu
from jax.experimental.pallas import tpu_sc as plsc

NPTS = 10000
DIM = 128
KNN = 16
NTILES = 32
QG = 4
GPT = 79
NVREG = NPTS // 16
CHUNK_V = 26
NCHUNK = 24
CBUF = CHUNK_V * 16


def _mm_body(x_ref, w1_ref, b1_ref, p_ref, q_ref):
    x = x_ref[...]
    w1a = w1_ref[:DIM, :]
    w1b = w1_ref[DIM:, :]
    q_ref[...] = jnp.dot(x, w1b, preferred_element_type=jnp.float32)
    p_ref[...] = (
        jnp.dot(x, w1a - w1b, preferred_element_type=jnp.float32) + b1_ref[...]
    )


def _phase_a(x, w1, b1r):
    return pl.pallas_call(
        _mm_body,
        out_shape=(
            jax.ShapeDtypeStruct((NPTS, DIM), jnp.float32),
            jax.ShapeDtypeStruct((NPTS, DIM), jnp.float32),
        ),
    )(x, w1, b1r)


def _bf16_rne(v):
    u = plsc.bitcast(v, jnp.uint32)
    r = (u + jnp.uint32(0x7FFF) + ((u >> jnp.uint32(16)) & jnp.uint32(1)))
    r = r & jnp.uint32(0xFFFF0000)
    return plsc.bitcast(r, jnp.float32)


def _knn_body(pos_hbm, q_hbm, g_hbm,
              bx, by, bz, sq, pflat,
              cd0, cj0, cd1, cj1, cd2, cj2, cd3, cj3,
              gbuf, gsem):
    cid = lax.axis_index("c")
    sid = lax.axis_index("s")
    wid = sid * 2 + cid
    cands = ((cd0, cj0), (cd1, cj1), (cd2, cj2), (cd3, cj3))

    pltpu.sync_copy(pos_hbm, pflat)
    iota3 = lax.iota(jnp.int32, 16) * 3

    def stage_body(i, _):
        base = i * 48
        gx = plsc.load_gather(pflat, [iota3 + base])
        gy = plsc.load_gather(pflat, [iota3 + (base + 1)])
        gz = plsc.load_gather(pflat, [iota3 + (base + 2)])
        sq[pl.ds(i * 16, 16)] = gx * gx + gy * gy + gz * gz
        bx[pl.ds(i * 16, 16)] = _bf16_rne(gx)
        by[pl.ds(i * 16, 16)] = _bf16_rne(gy)
        bz[pl.ds(i * 16, 16)] = _bf16_rne(gz)
        return 0

    lax.fori_loop(0, NVREG, stage_body, 0)

    iota = lax.iota(jnp.int32, 16)
    inf_v = jnp.full((16,), jnp.inf, jnp.float32)
    zero_i = jnp.zeros((16,), jnp.int32)
    negone_i = jnp.full((16,), -1, jnp.int32)

    def group_body(gi, _):
        nodes = [jnp.minimum(wid * (GPT * QG) + gi * QG + q, NPTS - 1)
                 for q in range(QG)]
        xis, yis, zis = [], [], []
        for q in range(QG):
            xis.append(bx[pl.ds(nodes[q], 16)][0] * -2.0)
            yis.append(by[pl.ds(nodes[q], 16)][0] * -2.0)
            zis.append(bz[pl.ds(nodes[q], 16)][0] * -2.0)

        def chunk_body(ch, carry):
            st = list(carry)
            base0 = (ch * CHUNK_V + 1) * 16
            for v in range(CHUNK_V):
                b = base0 + v * 16
                sqv = sq[pl.ds(b, 16)]
                bxv = bx[pl.ds(b, 16)]
                byv = by[pl.ds(b, 16)]
                bzv = bz[pl.ds(b, 16)]
                jv = iota + b
                for q in range(QG):
                    mvec = st[4 * q + 2]
                    curs = st[4 * q + 3]
                    d = (sqv + bxv * xis[q]) + (byv * yis[q] + bzv * zis[q])
                    hit = d < mvec
                    ps = plsc.cumsum(hit.astype(jnp.int32))
                    posn = curs + ps
                    plsc.store_scatter(cands[q][0], [posn], d, mask=hit)
                    plsc.store_scatter(cands[q][1], [posn], jv, mask=hit)
                    st[4 * q + 3] = curs + ps[15]

            for q in range(QG):
                topk = st[4 * q]
                topv = st[4 * q + 1]
                cnt_s = st[4 * q + 3][0] + 1
                node = nodes[q]
                cd, cj = cands[q]

                def do_merge(c2, cnt_s=cnt_s, node=node, cd=cd, cj=cj):
                    nv = (cnt_s + 15) >> 4

                    def mbody(it, c3):
                        tk, tv = c3
                        mb = it * 16
                        dc = cd[pl.ds(mb, 16)]
                        jc = cj[pl.ds(mb, 16)]
                        valid = iota < (cnt_s - mb)
                        dc = jnp.where(valid, dc, jnp.inf)
                        dc = jnp.where(jc == node, jnp.inf, dc)
                        sk, sv = plsc.sort_key_val(dc, jc)
                        rk = jnp.flip(sk)
                        rv = jnp.flip(sv)
                        take = rk < tk
                        nk = jnp.where(take, rk, tk)
                        nvv = jnp.where(take, rv, tv)
                        tk2, tv2 = plsc.sort_key_val(nk, nvv)
                        return (tk2, tv2)

                    return lax.fori_loop(0, nv, mbody, c2)

                topk, topv = lax.cond(cnt_s > 0, do_merge, lambda c2: c2,
                                      (topk, topv))
                st[4 * q] = topk
                st[4 * q + 1] = topv
                st[4 * q + 2] = jnp.broadcast_to(topk[15], (16,))
                st[4 * q + 3] = negone_i
            return tuple(st)

        sq0 = sq[pl.ds(0, 16)]
        bx0 = bx[pl.ds(0, 16)]
        by0 = by[pl.ds(0, 16)]
        bz0 = bz[pl.ds(0, 16)]
        seed = []
        for q in range(QG):
            d0 = (sq0 + bx0 * xis[q]) + (by0 * yis[q] + bz0 * zis[q])
            d0 = jnp.where(iota == nodes[q], jnp.inf, d0)
            sk, sv = plsc.sort_key_val(d0, iota)
            seed += [sk, sv, jnp.broadcast_to(sk[15], (16,)), negone_i]
        fin = lax.fori_loop(0, NCHUNK, chunk_body, tuple(seed))

        gd = [pltpu.async_copy(q_hbm.at[fin[4 * q + 1]], gbuf.at[q], gsem)
              for q in range(QG)]
        for c in gd:
            c.wait()
        sd = [pltpu.async_copy(gbuf.at[q], g_hbm.at[pl.ds(nodes[q] * KNN, KNN)],
                               gsem) for q in range(QG)]
        for c in sd:
            c.wait()
        return 0

    lax.fori_loop(0, GPT, group_body, 0)


def _sc_knn_gather(pos_flat, q):
    mesh = plsc.VectorSubcoreMesh(core_axis_name="c", subcore_axis_name="s")
    kern = pl.kernel(
        _knn_body,
        out_type=jax.ShapeDtypeStruct((NPTS * KNN, DIM), jnp.float32),
        mesh=mesh,
        scratch_types=[
            pltpu.VMEM((NPTS + 16,), jnp.float32),
            pltpu.VMEM((NPTS + 16,), jnp.float32),
            pltpu.VMEM((NPTS + 16,), jnp.float32),
            pltpu.VMEM((NPTS,), jnp.float32),
            pltpu.VMEM((3 * NPTS,), jnp.float32),
            pltpu.VMEM((CBUF,), jnp.float32),
            pltpu.VMEM((CBUF,), jnp.int32),
            pltpu.VMEM((CBUF,), jnp.float32),
            pltpu.VMEM((CBUF,), jnp.int32),
            pltpu.VMEM((CBUF,), jnp.float32),
            pltpu.VMEM((CBUF,), jnp.int32),
            pltpu.VMEM((CBUF,), jnp.float32),
            pltpu.VMEM((CBUF,), jnp.int32),
            pltpu.VMEM((QG, KNN, DIM), jnp.float32),
            pltpu.SemaphoreType.DMA,
        ],
        compiler_params=pltpu.CompilerParams(needs_layout_passes=False),
    )
    return kern(pos_flat, q)


NB = 256


def _mlp_body(g_ref, p_ref, w2_ref, b2_ref, o_ref):
    p = p_ref[...]
    w2 = w2_ref[...]
    acc = None
    for s in range(KNN):
        h1 = jnp.maximum(p + g_ref[:, s, :], 0.0)
        h2 = jnp.dot(h1, w2, preferred_element_type=jnp.float32)
        acc = h2 if acc is None else jnp.maximum(acc, h2)
    o_ref[...] = acc + b2_ref[...]


def _phase_c(ge, p, w2, b2r):
    grid = (pl.cdiv(NPTS, NB),)
    return pl.pallas_call(
        _mlp_body,
        grid=grid,
        in_specs=[
            pl.BlockSpec((NB, KNN, DIM), lambda i: (i, 0, 0)),
            pl.BlockSpec((NB, DIM), lambda i: (i, 0)),
            pl.BlockSpec((DIM, DIM), lambda i: (0, 0)),
            pl.BlockSpec((1, DIM), lambda i: (0, 0)),
        ],
        out_specs=pl.BlockSpec((NB, DIM), lambda i: (i, 0)),
        out_shape=jax.ShapeDtypeStruct((NPTS, DIM), jnp.float32),
    )(ge, p, w2, b2r)


def kernel(x, pos, W1, b1, W2, b2):
    p, q = _phase_a(x, W1, b1.reshape(1, DIM))
    g = _sc_knn_gather(pos.reshape(3 * NPTS), q)
    ge = g.reshape(NPTS, KNN, DIM)
    return _phase_c(ge, p, W2, b2.reshape(1, DIM))
